# baseline (speedup 1.0000x reference)
import functools
import math

import jax
import jax.numpy as jnp
from jax import lax
from jax.experimental import pallas as pl
from jax.experimental.pallas import tpu as pltpu
from jax.experimental.pallas import tpu_sc as plsc

_NG = 4
_NW = 2
_C = 128


@functools.cache
def _build(B, S, D, V):
    info = plsc.get_sparse_core_info()
    NC, NS, L = info.num_cores, info.num_subcores, info.num_lanes
    NWK = NC * NS
    R = B * S
    rows_w = R // NWK
    n_chunks = rows_w // _C
    n_outer = n_chunks // _NG
    scale = math.sqrt(float(D))

    mesh = plsc.VectorSubcoreMesh(core_axis_name="c", subcore_axis_name="s")

    @functools.partial(
        pl.kernel,
        out_type=jax.ShapeDtypeStruct((R, D), jnp.float32),
        mesh=mesh,
        scratch_types=[
            pltpu.VMEM((n_chunks, _C), jnp.int32),
            pltpu.MemorySpace.VMEM_SHARED((2 * S, D), jnp.float32),
            [pltpu.VMEM((_C, D), jnp.float32) for _ in range(_NG)],
            [pltpu.VMEM((_C, D), jnp.float32) for _ in range(_NW)],
            [pltpu.SemaphoreType.DMA for _ in range(_NG)],
            [pltpu.SemaphoreType.DMA for _ in range(_NG)],
            [pltpu.SemaphoreType.DMA for _ in range(_NW)],
        ],
    )
    def emb_kernel(table_hbm, x3_hbm, enc2_hbm, out_hbm,
                   idx_v, enc2_sh, gbufs, wbufs, psems, gsems, wsems):
        cid = lax.axis_index("c")
        sid = lax.axis_index("s")
        wid = sid * NC + cid
        row0 = wid * rows_w

        @pl.when(sid == 0)
        def _():
            pltpu.sync_copy(enc2_hbm, enc2_sh)
        pltpu.sync_copy(x3_hbm.at[wid], idx_v)
        plsc.subcore_barrier()

        def enc_src(p):
            off = lax.rem(p * _C, S)
            return enc2_sh.at[pl.ds(off, _C)]

        def out_dst(p):
            return out_hbm.at[pl.ds(row0 + p * _C, _C)]

        def prefill(p, g):
            pltpu.async_copy(enc_src(p), gbufs[g], psems[g])

        def gather_add(p, g):
            pltpu.async_copy(table_hbm.at[idx_v.at[p]], gbufs[g], gsems[g],
                             add=True)

        def wait_prefill(p, g):
            pltpu.make_async_copy(enc_src(p), gbufs[g], psems[g]).wait()

        def wait_gather(p, g):
            pltpu.make_async_copy(table_hbm.at[idx_v.at[p]], gbufs[g],
                                  gsems[g]).wait()

        def wait_writeout(p, w):
            pltpu.make_async_copy(wbufs[w], out_dst(p), wsems[w]).wait()

        for p in range(_NG):
            prefill(p, p)
        for p in range(2):
            wait_prefill(p, p)
            gather_add(p, p)

        def outer(i, carry):
            for g in range(_NG):
                q = i * _NG + g
                w = g % _NW
                g2 = (g + 2) % _NG

                @pl.when(q + 2 < n_chunks)
                def _():
                    wait_prefill(q + 2, g2)
                    gather_add(q + 2, g2)

                wait_gather(q, g)

                @pl.when(q >= _NW)
                def _():
                    wait_writeout(q - _NW, w)

                def row_body(r, c2):
                    for c in range(D // L):
                        sl = pl.ds(c * L, L)
                        wbufs[w][r, sl] = gbufs[g][r, sl] * scale
                    return c2

                lax.fori_loop(0, _C, row_body, 0, unroll=4)
                pltpu.async_copy(wbufs[w], out_dst(q), wsems[w])

                @pl.when(q + _NG < n_chunks)
                def _():
                    prefill(q + _NG, g)
            return carry

        lax.fori_loop(0, n_outer, outer, 0)
        for p in range(n_chunks - _NW, n_chunks):
            wait_writeout(p, p % _NW)

    return emb_kernel


def kernel(x, table, encoding):
    B, S = x.shape
    V, D = table.shape
    NWK = 32
    rows_w = B * S // NWK
    x3 = x.astype(jnp.int32).reshape(NWK, rows_w // _C, _C)
    enc = encoding[:S, :] * (1.0 / math.sqrt(float(D)))
    enc2 = jnp.concatenate([enc, enc], axis=0)
    out = _build(B, S, D, V)(table, x3, enc2)
    return out.reshape(B, S, D)


# speedup vs baseline: 2.9581x; 2.9581x over previous
import functools
import math

import jax
import jax.numpy as jnp
from jax import lax
from jax.experimental import pallas as pl
from jax.experimental.pallas import tpu as pltpu
from jax.experimental.pallas import tpu_sc as plsc

_NBUF = 5
_C = 128


@functools.cache
def _build(B, S, D, V):
    info = plsc.get_sparse_core_info()
    NC, NS, L = info.num_cores, info.num_subcores, info.num_lanes
    NW = NC * NS
    R = B * S
    rows_w = R // NW
    n_chunks = rows_w // _C
    n_outer = n_chunks // _NBUF
    scale = math.sqrt(float(D))

    mesh = plsc.VectorSubcoreMesh(core_axis_name="c", subcore_axis_name="s")

    @functools.partial(
        pl.kernel,
        out_type=jax.ShapeDtypeStruct((R, D), jnp.float32),
        mesh=mesh,
        scratch_types=[
            pltpu.VMEM((n_chunks, _C), jnp.int32),
            pltpu.MemorySpace.VMEM_SHARED((2 * S, D), jnp.float32),
            [pltpu.VMEM((_C, D), jnp.float32) for _ in range(_NBUF)],
            [pltpu.SemaphoreType.DMA for _ in range(_NBUF)],
            [pltpu.SemaphoreType.DMA for _ in range(_NBUF)],
            [pltpu.SemaphoreType.DMA for _ in range(_NBUF)],
        ],
    )
    def emb_kernel(table_hbm, x3_hbm, enc2_hbm, out_hbm,
                   idx_v, enc2_sh, bufs, psems, gsems, wsems):
        cid = lax.axis_index("c")
        sid = lax.axis_index("s")
        wid = sid * NC + cid
        row0 = wid * rows_w

        @pl.when(sid == 0)
        def _():
            pltpu.sync_copy(enc2_hbm, enc2_sh)
        pltpu.sync_copy(x3_hbm.at[wid], idx_v)
        plsc.subcore_barrier()

        def enc_src(p):
            off = lax.rem(p * _C, S)
            return enc2_sh.at[pl.ds(off, _C)]

        def out_dst(p):
            return out_hbm.at[pl.ds(row0 + p * _C, _C)]

        def prefill(p, b):
            pltpu.async_copy(enc_src(p), bufs[b], psems[b])

        def gather_add(p, b):
            pltpu.async_copy(table_hbm.at[idx_v.at[p]], bufs[b], gsems[b],
                             add=True)

        def wait_prefill(p, b):
            pltpu.make_async_copy(enc_src(p), bufs[b], psems[b]).wait()

        def wait_gather(p, b):
            pltpu.make_async_copy(table_hbm.at[idx_v.at[p]], bufs[b],
                                  gsems[b]).wait()

        def wait_writeout(p, b):
            pltpu.make_async_copy(bufs[b], out_dst(p), wsems[b]).wait()

        for p in range(3):
            prefill(p, p)
        for p in range(2):
            wait_prefill(p, p)
            gather_add(p, p)

        def outer(i, carry):
            for b in range(_NBUF):
                q = i * _NBUF + b
                qa = q + 3
                ba = (b + 3) % _NBUF

                @pl.when(qa < n_chunks)
                def _():
                    @pl.when(q >= 2)
                    def _():
                        wait_writeout(qa - _NBUF, ba)
                    prefill(qa, ba)

                qb = q + 2
                bb = (b + 2) % _NBUF

                @pl.when(qb < n_chunks)
                def _():
                    wait_prefill(qb, bb)
                    gather_add(qb, bb)

                wait_gather(q, b)

                def row_body(r, c2):
                    for c in range(D // L):
                        sl = pl.ds(c * L, L)
                        bufs[b][r, sl] = bufs[b][r, sl] * scale
                    return c2

                lax.fori_loop(0, _C, row_body, 0, unroll=2)
                pltpu.async_copy(bufs[b], out_dst(q), wsems[b])
            return carry

        lax.fori_loop(0, n_outer, outer, 0)
        for p in range(n_chunks - _NBUF, n_chunks):
            wait_writeout(p, p % _NBUF)

    return emb_kernel


def kernel(x, table, encoding):
    B, S = x.shape
    V, D = table.shape
    NW = 32
    rows_w = B * S // NW
    x3 = x.astype(jnp.int32).reshape(NW, rows_w // _C, _C)
    enc = encoding[:S, :] * (1.0 / math.sqrt(float(D)))
    enc2 = jnp.concatenate([enc, enc], axis=0)
    out = _build(B, S, D, V)(table, x3, enc2)
    return out.reshape(B, S, D)


# speedup vs baseline: 3.0027x; 1.0151x over previous
import functools
import math

import jax
import jax.numpy as jnp
from jax import lax
from jax.experimental import pallas as pl
from jax.experimental.pallas import tpu as pltpu
from jax.experimental.pallas import tpu_sc as plsc

_NBUF = 4
_H = 100


@functools.cache
def _build(B, S, D, V):
    info = plsc.get_sparse_core_info()
    NC, NS, L = info.num_cores, info.num_subcores, info.num_lanes
    NW = NC * NS
    R = B * S
    rows_w = R // NW
    n_chunks = rows_w // S
    n_outer = n_chunks // _NBUF
    scale = math.sqrt(float(D))

    mesh = plsc.VectorSubcoreMesh(core_axis_name="c", subcore_axis_name="s")

    @functools.partial(
        pl.kernel,
        out_type=jax.ShapeDtypeStruct((R, D), jnp.float32),
        mesh=mesh,
        scratch_types=[
            [pltpu.VMEM((2, _H), jnp.int32) for _ in range(_NBUF)],
            pltpu.MemorySpace.VMEM_SHARED((S, D), jnp.float32),
            [pltpu.VMEM((S, D), jnp.float32) for _ in range(_NBUF)],
            [pltpu.SemaphoreType.DMA for _ in range(_NBUF)],
            [pltpu.SemaphoreType.DMA for _ in range(_NBUF)],
            [pltpu.SemaphoreType.DMA for _ in range(_NBUF)],
            [pltpu.SemaphoreType.DMA for _ in range(_NBUF)],
        ],
    )
    def emb_kernel(table_hbm, x4_hbm, enc_hbm, out_hbm,
                   ibufs, enc_sh, bufs, isems, psems, gsems, wsems):
        cid = lax.axis_index("c")
        sid = lax.axis_index("s")
        wid = sid * NC + cid
        row0 = wid * rows_w

        @pl.when(sid == 0)
        def _():
            pltpu.sync_copy(enc_hbm, enc_sh)
        plsc.subcore_barrier()

        def out_dst(p):
            return out_hbm.at[pl.ds(row0 + p * S, S)]

        def idx_fetch(p, b):
            pltpu.async_copy(x4_hbm.at[wid, p], ibufs[b], isems[b])

        def wait_idx(p, b):
            pltpu.make_async_copy(x4_hbm.at[wid, p], ibufs[b], isems[b]).wait()

        def prefill(p, b):
            pltpu.async_copy(enc_sh, bufs[b], psems[b])

        def gather_add(p, b):
            for h in range(2):
                pltpu.async_copy(table_hbm.at[ibufs[b].at[h]],
                                 bufs[b].at[pl.ds(h * _H, _H)], gsems[b],
                                 add=True)

        def wait_prefill(p, b):
            pltpu.make_async_copy(enc_sh, bufs[b], psems[b]).wait()

        def wait_gather(p, b):
            for h in range(2):
                pltpu.make_async_copy(table_hbm.at[ibufs[b].at[h]],
                                      bufs[b].at[pl.ds(h * _H, _H)],
                                      gsems[b]).wait()

        def wait_writeout(p, b):
            pltpu.make_async_copy(bufs[b], out_dst(p), wsems[b]).wait()

        for p in range(3):
            idx_fetch(p, p)
        for p in range(2):
            prefill(p, p)
        wait_idx(0, 0)
        wait_prefill(0, 0)
        gather_add(0, 0)

        def outer(i, carry):
            for b in range(_NBUF):
                q = i * _NBUF + b
                qa = q + 2
                ba = (b + 2) % _NBUF
                qi = q + 3
                bi = (b + 3) % _NBUF

                @pl.when(qi < n_chunks)
                def _():
                    idx_fetch(qi, bi)

                @pl.when(qa < n_chunks)
                def _():
                    @pl.when(q >= 2)
                    def _():
                        wait_writeout(qa - _NBUF, ba)
                    prefill(qa, ba)

                qb = q + 1
                bb = (b + 1) % _NBUF

                @pl.when(qb < n_chunks)
                def _():
                    wait_idx(qb, bb)
                    wait_prefill(qb, bb)
                    gather_add(qb, bb)

                wait_gather(q, b)

                def row_body(r, c2):
                    for c in range(D // L):
                        sl = pl.ds(c * L, L)
                        bufs[b][r, sl] = bufs[b][r, sl] * scale
                    return c2

                lax.fori_loop(0, S, row_body, 0, unroll=2)
                pltpu.async_copy(bufs[b], out_dst(q), wsems[b])
            return carry

        lax.fori_loop(0, n_outer, outer, 0)
        for p in range(n_chunks - _NBUF, n_chunks):
            wait_writeout(p, p % _NBUF)

    return emb_kernel


def kernel(x, table, encoding):
    B, S = x.shape
    V, D = table.shape
    NW = 32
    rows_w = B * S // NW
    x4 = x.astype(jnp.int32).reshape(NW, rows_w // S, 2, _H)
    enc = encoding[:S, :] * (1.0 / math.sqrt(float(D)))
    out = _build(B, S, D, V)(table, x4, enc)
    return out.reshape(B, S, D)
